# baseline (speedup 1.0000x reference)
import functools

import jax
import jax.numpy as jnp
from jax import lax
from jax.experimental import pallas as pl
from jax.experimental.pallas import tpu as pltpu
from jax.experimental.pallas import tpu_sc as plsc

_GAMMA = 24.0
_NC = 2
_NS = 16
_NW = _NC * _NS
_L = 16
_B = 4096
_NNEG = 128
_D = 64
_W = 2 * _D
_BW = _B // _NW
_NBUF = 2


def _pre(idx_v, half_v, ofs_v, n):
    def body(i, _):
        sl = pl.ds(i * _L, _L)
        v = idx_v[sl]
        half_v[sl] = jax.lax.shift_right_logical(v, 1)
        ofs_v[sl] = jax.lax.shift_left(jnp.bitwise_and(v, 1), 6)
        return 0
    lax.fori_loop(0, n // _L, body, 0)


def _gqe_body(ent, rel, pos_i, neg_i, qe_i, qr_i,
              pos_out, neg_out,
              qe_v, qr_v, pos_v, qe_o, qr_o, pos_o,
              neg_idx_v, neg_ofs_v, q_final,
              bufs, out_pos_v, out_neg_v,
              sem_a, sem_b, neg_sems):
    wid = lax.axis_index("s") * _NC + lax.axis_index("c")
    base = wid * _BW
    iota = lax.iota(jnp.int32, _L)

    pltpu.sync_copy(qe_i.at[pl.ds(base, _BW)], qe_v)
    pltpu.sync_copy(qr_i.at[pl.ds(base, _BW)], qr_v)
    pltpu.sync_copy(pos_i.at[pl.ds(base, _BW)], pos_v)
    pltpu.sync_copy(neg_i.at[pl.ds(base, _BW)], neg_idx_v)

    _pre(qe_v, qe_v, qe_o, _BW)
    _pre(qr_v, qr_v, qr_o, _BW)
    _pre(pos_v, pos_v, pos_o, _BW)

    def npre(b, _):
        def chunk(i, _):
            sl = pl.ds(i * _L, _L)
            v = neg_idx_v[b, sl]
            neg_idx_v[b, sl] = jax.lax.shift_right_logical(v, 1)
            neg_ofs_v[b, sl] = jax.lax.shift_left(jnp.bitwise_and(v, 1), 6)
            return 0
        lax.fori_loop(0, _NNEG // _L, chunk, 0)
        return 0
    lax.fori_loop(0, _BW, npre, 0)

    ca = pltpu.make_async_copy(ent.at[qe_v], bufs[0], sem_a)
    cb = pltpu.make_async_copy(rel.at[qr_v], bufs[1], sem_b)
    ca.start()
    cb.start()
    ca.wait()
    cb.wait()

    def qbody(b, _):
        bb = jnp.full((_L,), b, jnp.int32)
        eo = plsc.load_gather(qe_o, [bb])
        ro = plsc.load_gather(qr_o, [bb])
        for s in range(_D // _L):
            col = s * _L + iota
            ev = plsc.load_gather(bufs[0], [bb, eo + col])
            rv = plsc.load_gather(bufs[1], [bb, ro + col])
            q_final[b, pl.ds(s * _L, _L)] = ev + rv
        return 0
    lax.fori_loop(0, _BW, qbody, 0)

    cp = pltpu.make_async_copy(ent.at[pos_v], bufs[0], sem_a)
    cp.start()
    cp.wait()

    for g in range(_BW // _L):
        b_ids = g * _L + iota
        po = plsc.load_gather(pos_o, [b_ids])

        def pbody(d, acc):
            col = jnp.bitwise_and(d + iota, _D - 1)
            vals = plsc.load_gather(bufs[0], [b_ids, po + col])
            qv = plsc.load_gather(q_final, [b_ids, col])
            return acc + jnp.abs(vals - qv)

        acc = lax.fori_loop(0, _D, pbody, jnp.zeros((_L,), jnp.float32),
                            unroll=4)
        out_pos_v[pl.ds(g * _L, _L)] = _GAMMA - acc

    n_grp = _NNEG // _L
    grp_rows = [g * _L + iota for g in range(n_grp)]

    def neg_copy(b, k):
        return pltpu.make_async_copy(ent.at[neg_idx_v.at[b]], bufs[k],
                                     neg_sems[k])

    for k in range(_NBUF):
        neg_copy(k, k).start()

    def compute_batch(b, buf):
        bb = jnp.full((_L,), b, jnp.int32)
        nofs = [plsc.load_gather(neg_ofs_v, [bb, grp_rows[g]])
                for g in range(n_grp)]

        def nbody(d, accs):
            col = jnp.bitwise_and(d + iota, _D - 1)
            qb = plsc.load_gather(q_final, [bb, col])
            out = []
            for g in range(n_grp):
                vals = plsc.load_gather(buf, [grp_rows[g], nofs[g] + col])
                out.append(accs[g] + jnp.abs(vals - qb))
            return tuple(out)

        accs = lax.fori_loop(
            0, _D, nbody,
            tuple(jnp.zeros((_L,), jnp.float32) for _ in range(n_grp)),
            unroll=2)
        for g in range(n_grp):
            out_neg_v[b, pl.ds(g * _L, _L)] = _GAMMA - accs[g]

    def outer(it, _):
        for k in range(_NBUF):
            b = it * _NBUF + k
            neg_copy(b, k).wait()
            compute_batch(b, bufs[k])

            @pl.when(b + _NBUF < _BW)
            def _():
                neg_copy(b + _NBUF, k).start()
        return 0

    lax.fori_loop(0, _BW // _NBUF, outer, 0)

    pltpu.sync_copy(out_pos_v, pos_out.at[pl.ds(base, _BW)])
    pltpu.sync_copy(out_neg_v, neg_out.at[pl.ds(base, _BW)])


_NE = 1000000
_NCHUNK = _NE // _W
_TRIPS = 248


def _tr_body(src, tail, out, bufs, sts, tailbuf, in_sems, out_sems):
    wid = lax.axis_index("s") * _NC + lax.axis_index("c")
    iota = lax.iota(jnp.int32, _L)

    def chunk_of(it):
        c = wid + it * _NW
        return jnp.where(c < _NCHUNK, c, wid)

    def in_copy(it, k):
        c = chunk_of(it)
        return pltpu.make_async_copy(src.at[:, pl.ds(c * _W, _W)], bufs[k],
                                     in_sems[k])

    def out_copy(it, k):
        c = chunk_of(it)
        return pltpu.make_async_copy(sts[k], out.at[pl.ds(c * _D, _D)],
                                     out_sems[k])

    for k in range(4):
        in_copy(k, k).start()

    grp_e = [g * _L + iota for g in range(_W // _L)]
    grp_row = [jax.lax.shift_right_logical(e, 1) for e in grp_e]
    grp_par = [jax.lax.shift_left(jnp.bitwise_and(e, 1), 6) for e in grp_e]

    def transpose(k):
        def tbody(t, _):
            dvec = jnp.bitwise_and(t + iota, _D - 1)
            for g in range(_W // _L):
                val = plsc.load_gather(bufs[k], [dvec, grp_e[g]])
                plsc.store_scatter(sts[k], [grp_row[g], grp_par[g] + dvec],
                                   val)
            return 0
        lax.fori_loop(0, _D, tbody, 0, unroll=2)

    def step(j, _):
        for k in range(4):
            it = 4 * j + k
            in_copy(it, k).wait()

            @pl.when(j > 0)
            def _():
                out_copy(it - 4, k).wait()

            transpose(k)
            out_copy(it, k).start()

            @pl.when(it + 4 < _TRIPS)
            def _():
                in_copy(it + 4, k).start()
        return 0

    lax.fori_loop(0, _TRIPS // 4, step, 0)
    for k in range(4):
        out_copy(_TRIPS - 4 + k, k).wait()

    @pl.when(wid == 0)
    def _():
        pltpu.sync_copy(tail, tailbuf)
        pltpu.sync_copy(tailbuf, out.at[pl.ds(_NCHUNK * _D, _D // 2)])


@functools.cache
def _build_tr():
    mesh = plsc.VectorSubcoreMesh(core_axis_name="c", subcore_axis_name="s")
    scratch = [
        [pltpu.VMEM((_D, _W), jnp.float32) for _ in range(4)],
        [pltpu.VMEM((_D, _W), jnp.float32) for _ in range(4)],
        pltpu.VMEM((_D // 2, _W), jnp.float32),
        [pltpu.SemaphoreType.DMA for _ in range(4)],
        [pltpu.SemaphoreType.DMA for _ in range(4)],
    ]
    return pl.kernel(
        _tr_body,
        out_type=jax.ShapeDtypeStruct((_NE // 2, _W), jnp.float32),
        mesh=mesh,
        scratch_types=scratch,
        compiler_params=pltpu.CompilerParams(needs_layout_passes=False),
    )


@functools.cache
def _build():
    mesh = plsc.VectorSubcoreMesh(core_axis_name="c", subcore_axis_name="s")
    scratch = [
        pltpu.VMEM((_BW,), jnp.int32),
        pltpu.VMEM((_BW,), jnp.int32),
        pltpu.VMEM((_BW,), jnp.int32),
        pltpu.VMEM((_BW,), jnp.int32),
        pltpu.VMEM((_BW,), jnp.int32),
        pltpu.VMEM((_BW,), jnp.int32),
        pltpu.VMEM((_BW, _NNEG), jnp.int32),
        pltpu.VMEM((_BW, _NNEG), jnp.int32),
        pltpu.VMEM((_BW, _W), jnp.float32),
        [pltpu.VMEM((_NNEG, _W), jnp.float32) for _ in range(_NBUF)],
        pltpu.VMEM((_BW,), jnp.float32),
        pltpu.VMEM((_BW, _NNEG), jnp.float32),
        pltpu.SemaphoreType.DMA,
        pltpu.SemaphoreType.DMA,
        [pltpu.SemaphoreType.DMA for _ in range(_NBUF)],
    ]
    return pl.kernel(
        _gqe_body,
        out_type=(
            jax.ShapeDtypeStruct((_B,), jnp.float32),
            jax.ShapeDtypeStruct((_B, _NNEG), jnp.float32),
        ),
        mesh=mesh,
        scratch_types=scratch,
        compiler_params=pltpu.CompilerParams(needs_layout_passes=False),
    )


def kernel(entity_table, relation_table, positive_sample, negative_sample,
           q_entity, q_relation):
    tail = jnp.reshape(entity_table[_NCHUNK * _W:], (_D // 2, _W))
    ent2 = _build_tr()(entity_table.T, tail)
    rel2 = jnp.reshape(relation_table, (relation_table.shape[0] // 2, _W))
    return _build()(ent2, rel2, positive_sample, negative_sample,
                    q_entity, q_relation)


# speedup vs baseline: 1.0109x; 1.0109x over previous
import functools

import jax
import jax.numpy as jnp
from jax import lax
from jax.experimental import pallas as pl
from jax.experimental.pallas import tpu as pltpu
from jax.experimental.pallas import tpu_sc as plsc

_GAMMA = 24.0
_NC = 2
_NS = 16
_NW = _NC * _NS
_L = 16
_B = 4096
_NNEG = 128
_D = 64
_W = 2 * _D
_BW = _B // _NW
_NBUF = 2


def _pre(idx_v, half_v, ofs_v, n):
    def body(i, _):
        sl = pl.ds(i * _L, _L)
        v = idx_v[sl]
        half_v[sl] = jax.lax.shift_right_logical(v, 1)
        ofs_v[sl] = jax.lax.shift_left(jnp.bitwise_and(v, 1), 6)
        return 0
    lax.fori_loop(0, n // _L, body, 0)


def _gqe_body(ent, rel, pos_i, neg_i, qe_i, qr_i,
              pos_out, neg_out,
              qe_v, qr_v, pos_v, qe_o, qr_o, pos_o,
              neg_idx_v, neg_ofs_v, q_final,
              bufs, out_pos_v, out_neg_v,
              sem_a, sem_b, neg_sems):
    wid = lax.axis_index("s") * _NC + lax.axis_index("c")
    base = wid * _BW
    iota = lax.iota(jnp.int32, _L)

    pltpu.sync_copy(qe_i.at[pl.ds(base, _BW)], qe_v)
    pltpu.sync_copy(qr_i.at[pl.ds(base, _BW)], qr_v)
    pltpu.sync_copy(pos_i.at[pl.ds(base, _BW)], pos_v)
    pltpu.sync_copy(neg_i.at[pl.ds(base, _BW)], neg_idx_v)

    _pre(qe_v, qe_v, qe_o, _BW)
    _pre(qr_v, qr_v, qr_o, _BW)
    _pre(pos_v, pos_v, pos_o, _BW)

    def npre(b, _):
        def chunk(i, _):
            sl = pl.ds(i * _L, _L)
            v = neg_idx_v[b, sl]
            neg_idx_v[b, sl] = jax.lax.shift_right_logical(v, 1)
            neg_ofs_v[b, sl] = jax.lax.shift_left(jnp.bitwise_and(v, 1), 6)
            return 0
        lax.fori_loop(0, _NNEG // _L, chunk, 0)
        return 0
    lax.fori_loop(0, _BW, npre, 0)

    ca = pltpu.make_async_copy(ent.at[qe_v], bufs[0], sem_a)
    cb = pltpu.make_async_copy(rel.at[qr_v], bufs[1], sem_b)
    ca.start()
    cb.start()
    ca.wait()
    cb.wait()

    def qbody(b, _):
        bb = jnp.full((_L,), b, jnp.int32)
        eo = plsc.load_gather(qe_o, [bb])
        ro = plsc.load_gather(qr_o, [bb])
        for s in range(_D // _L):
            col = s * _L + iota
            ev = plsc.load_gather(bufs[0], [bb, eo + col])
            rv = plsc.load_gather(bufs[1], [bb, ro + col])
            q_final[b, pl.ds(s * _L, _L)] = ev + rv
        return 0
    lax.fori_loop(0, _BW, qbody, 0)

    cp = pltpu.make_async_copy(ent.at[pos_v], bufs[0], sem_a)
    cp.start()
    cp.wait()

    for g in range(_BW // _L):
        b_ids = g * _L + iota
        po = plsc.load_gather(pos_o, [b_ids])

        def pbody(d, acc):
            col = jnp.bitwise_and(d + iota, _D - 1)
            vals = plsc.load_gather(bufs[0], [b_ids, po + col])
            qv = plsc.load_gather(q_final, [b_ids, col])
            return acc + jnp.abs(vals - qv)

        acc = lax.fori_loop(0, _D, pbody, jnp.zeros((_L,), jnp.float32),
                            unroll=4)
        out_pos_v[pl.ds(g * _L, _L)] = _GAMMA - acc

    n_grp = _NNEG // _L
    grp_rows = [g * _L + iota for g in range(n_grp)]

    def neg_copy(b, k):
        return pltpu.make_async_copy(ent.at[neg_idx_v.at[b]], bufs[k],
                                     neg_sems[k])

    for k in range(_NBUF):
        neg_copy(k, k).start()

    def compute_batch(b, buf):
        bb = jnp.full((_L,), b, jnp.int32)
        nofs = [plsc.load_gather(neg_ofs_v, [bb, grp_rows[g]])
                for g in range(n_grp)]

        def nbody(d, accs):
            col = jnp.bitwise_and(d + iota, _D - 1)
            qb = plsc.load_gather(q_final, [bb, col])
            out = []
            for g in range(n_grp):
                vals = plsc.load_gather(buf, [grp_rows[g], nofs[g] + col])
                out.append(accs[g] + jnp.abs(vals - qb))
            return tuple(out)

        accs = lax.fori_loop(
            0, _D, nbody,
            tuple(jnp.zeros((_L,), jnp.float32) for _ in range(n_grp)),
            unroll=2)
        for g in range(n_grp):
            out_neg_v[b, pl.ds(g * _L, _L)] = _GAMMA - accs[g]

    def outer(it, _):
        for k in range(_NBUF):
            b = it * _NBUF + k
            neg_copy(b, k).wait()
            compute_batch(b, bufs[k])

            @pl.when(b + _NBUF < _BW)
            def _():
                neg_copy(b + _NBUF, k).start()
        return 0

    lax.fori_loop(0, _BW // _NBUF, outer, 0)

    pltpu.sync_copy(out_pos_v, pos_out.at[pl.ds(base, _BW)])
    pltpu.sync_copy(out_neg_v, neg_out.at[pl.ds(base, _BW)])


_NE = 1000000
_CW = 256
_NCHUNK = _NE // _CW
_TRIPS = 123


def _tr_body(src, tail, out, bufs, sts, tailbuf, in_sems, out_sems):
    wid = lax.axis_index("s") * _NC + lax.axis_index("c")
    iota = lax.iota(jnp.int32, _L)

    def chunk_of(it):
        c = wid + it * _NW
        return jnp.where(c < _NCHUNK, c, wid)

    def in_copy(it, k):
        c = chunk_of(it)
        return pltpu.make_async_copy(src.at[:, pl.ds(c * _CW, _CW)], bufs[k],
                                     in_sems[k])

    def out_copy(it, k):
        c = chunk_of(it)
        return pltpu.make_async_copy(sts[k],
                                     out.at[pl.ds(c * (_CW // 2), _CW // 2)],
                                     out_sems[k])

    for k in range(3):
        in_copy(k, k).start()

    grp_e = [g * _L + iota for g in range(_CW // _L)]
    grp_row = [jax.lax.shift_right_logical(e, 1) for e in grp_e]
    grp_par = [jax.lax.shift_left(jnp.bitwise_and(e, 1), 6) for e in grp_e]

    def transpose(k):
        def tbody(t, _):
            dvec = jnp.bitwise_and(t + iota, _D - 1)
            for g in range(_CW // _L):
                val = plsc.load_gather(bufs[k], [dvec, grp_e[g]])
                plsc.store_scatter(sts[k], [grp_row[g], grp_par[g] + dvec],
                                   val)
            return 0
        lax.fori_loop(0, _D, tbody, 0, unroll=2)

    def step(j, _):
        for k in range(3):
            it = 3 * j + k
            in_copy(it, k).wait()

            @pl.when(j > 0)
            def _():
                out_copy(it - 3, k).wait()

            transpose(k)
            out_copy(it, k).start()

            @pl.when(it + 3 < _TRIPS)
            def _():
                in_copy(it + 3, k).start()
        return 0

    lax.fori_loop(0, _TRIPS // 3, step, 0)
    for k in range(3):
        out_copy(_TRIPS - 3 + k, k).wait()

    @pl.when(wid == 0)
    def _():
        pltpu.sync_copy(tail, tailbuf)
        pltpu.sync_copy(tailbuf, out.at[pl.ds(_NCHUNK * (_CW // 2),
                                              _D // 2)])


@functools.cache
def _build_tr():
    mesh = plsc.VectorSubcoreMesh(core_axis_name="c", subcore_axis_name="s")
    scratch = [
        [pltpu.VMEM((_D, _CW), jnp.float32) for _ in range(3)],
        [pltpu.VMEM((_CW // 2, _W), jnp.float32) for _ in range(3)],
        pltpu.VMEM((_D // 2, _W), jnp.float32),
        [pltpu.SemaphoreType.DMA for _ in range(3)],
        [pltpu.SemaphoreType.DMA for _ in range(3)],
    ]
    return pl.kernel(
        _tr_body,
        out_type=jax.ShapeDtypeStruct((_NE // 2, _W), jnp.float32),
        mesh=mesh,
        scratch_types=scratch,
        compiler_params=pltpu.CompilerParams(needs_layout_passes=False),
    )


@functools.cache
def _build():
    mesh = plsc.VectorSubcoreMesh(core_axis_name="c", subcore_axis_name="s")
    scratch = [
        pltpu.VMEM((_BW,), jnp.int32),
        pltpu.VMEM((_BW,), jnp.int32),
        pltpu.VMEM((_BW,), jnp.int32),
        pltpu.VMEM((_BW,), jnp.int32),
        pltpu.VMEM((_BW,), jnp.int32),
        pltpu.VMEM((_BW,), jnp.int32),
        pltpu.VMEM((_BW, _NNEG), jnp.int32),
        pltpu.VMEM((_BW, _NNEG), jnp.int32),
        pltpu.VMEM((_BW, _W), jnp.float32),
        [pltpu.VMEM((_NNEG, _W), jnp.float32) for _ in range(_NBUF)],
        pltpu.VMEM((_BW,), jnp.float32),
        pltpu.VMEM((_BW, _NNEG), jnp.float32),
        pltpu.SemaphoreType.DMA,
        pltpu.SemaphoreType.DMA,
        [pltpu.SemaphoreType.DMA for _ in range(_NBUF)],
    ]
    return pl.kernel(
        _gqe_body,
        out_type=(
            jax.ShapeDtypeStruct((_B,), jnp.float32),
            jax.ShapeDtypeStruct((_B, _NNEG), jnp.float32),
        ),
        mesh=mesh,
        scratch_types=scratch,
        compiler_params=pltpu.CompilerParams(needs_layout_passes=False),
    )


def kernel(entity_table, relation_table, positive_sample, negative_sample,
           q_entity, q_relation):
    tail = jnp.reshape(entity_table[_NCHUNK * _CW:], (_D // 2, _W))
    ent2 = _build_tr()(entity_table.T, tail)
    rel2 = jnp.reshape(relation_table, (relation_table.shape[0] // 2, _W))
    return _build()(ent2, rel2, positive_sample, negative_sample,
                    q_entity, q_relation)


# speedup vs baseline: 1.1375x; 1.1252x over previous
import functools

import jax
import jax.numpy as jnp
from jax import lax
from jax.experimental import pallas as pl
from jax.experimental.pallas import tpu as pltpu
from jax.experimental.pallas import tpu_sc as plsc

_GAMMA = 24.0
_NC = 2
_NS = 16
_NW = _NC * _NS
_L = 16
_B = 4096
_NNEG = 128
_D = 64
_BW = _B // _NW
_NBUF = 4

_NE = 1000000
_CW = 256
_NCHUNK = _NE // _CW
_NTAIL = _NE - _NCHUNK * _CW
_TRIPS = 123


def _tr_body(src, tail, out, bufs, sts, tailbuf, in_sems, out_sems):
    wid = lax.axis_index("s") * _NC + lax.axis_index("c")
    iota = lax.iota(jnp.int32, _L)

    def chunk_of(it):
        c = wid + it * _NW
        return jnp.where(c < _NCHUNK, c, wid)

    def in_copy(it, k):
        c = chunk_of(it)
        return pltpu.make_async_copy(src.at[:, pl.ds(c * _CW, _CW)], bufs[k],
                                     in_sems[k])

    def out_copy(it, k):
        c = chunk_of(it)
        return pltpu.make_async_copy(sts[k],
                                     out.at[pl.ds(c * _CW * _D, _CW * _D)],
                                     out_sems[k])

    for k in range(3):
        in_copy(k, k).start()

    grp_e = [g * _L + iota for g in range(_CW // _L)]
    grp_base = [jax.lax.shift_left(e, 6) for e in grp_e]

    def transpose(k):
        def tbody(t, _):
            dvec = jnp.bitwise_and(t + iota, _D - 1)
            for g in range(_CW // _L):
                val = plsc.load_gather(bufs[k], [dvec, grp_e[g]])
                plsc.store_scatter(sts[k], [grp_base[g] + dvec], val)
            return 0
        lax.fori_loop(0, _D, tbody, 0, unroll=2)

    def step(j, _):
        for k in range(3):
            it = 3 * j + k
            in_copy(it, k).wait()

            @pl.when(j > 0)
            def _():
                out_copy(it - 3, k).wait()

            transpose(k)
            out_copy(it, k).start()

            @pl.when(it + 3 < _TRIPS)
            def _():
                in_copy(it + 3, k).start()
        return 0

    lax.fori_loop(0, _TRIPS // 3, step, 0)
    for k in range(3):
        out_copy(_TRIPS - 3 + k, k).wait()

    @pl.when(wid == 0)
    def _():
        pltpu.sync_copy(tail, tailbuf)
        pltpu.sync_copy(tailbuf,
                        out.at[pl.ds(_NCHUNK * _CW * _D, _NTAIL * _D)])


@functools.cache
def _build_tr():
    mesh = plsc.VectorSubcoreMesh(core_axis_name="c", subcore_axis_name="s")
    scratch = [
        [pltpu.VMEM((_D, _CW), jnp.float32) for _ in range(3)],
        [pltpu.VMEM((_CW * _D,), jnp.float32) for _ in range(3)],
        pltpu.VMEM((_NTAIL * _D,), jnp.float32),
        [pltpu.SemaphoreType.DMA for _ in range(3)],
        [pltpu.SemaphoreType.DMA for _ in range(3)],
    ]
    return pl.kernel(
        _tr_body,
        out_type=jax.ShapeDtypeStruct((_NE * _D,), jnp.float32),
        mesh=mesh,
        scratch_types=scratch,
        compiler_params=pltpu.CompilerParams(needs_layout_passes=False),
    )


def _gqe_body(ent, rel, pos_i, neg_i, qe_i, qr_i,
              pos_out, neg_out,
              qe_v, qr_v, pos_idx_v, neg_idx_v,
              q_rows, r_rows, pos_rows,
              bufs, out_pos_v, out_neg_v,
              sem_q, sem_r, sem_p, neg_sems):
    wid = lax.axis_index("s") * _NC + lax.axis_index("c")
    base = wid * _BW
    iota = lax.iota(jnp.int32, _L)

    pltpu.sync_copy(qe_i.at[pl.ds(base, _BW)], qe_v)
    pltpu.sync_copy(qr_i.at[pl.ds(base, _BW)], qr_v)
    pltpu.sync_copy(pos_i.at[pl.ds(base, _BW)], pos_idx_v)
    pltpu.sync_copy(neg_i.at[pl.ds(base, _BW)], neg_idx_v)

    cq = pltpu.make_async_copy(ent.at[qe_v], q_rows, sem_q)
    cr = pltpu.make_async_copy(rel.at[qr_v], r_rows, sem_r)
    cp = pltpu.make_async_copy(ent.at[pos_idx_v], pos_rows, sem_p)
    cq.start()
    cr.start()
    cp.start()

    def neg_copy(b, k):
        return pltpu.make_async_copy(ent.at[neg_idx_v.at[b]], bufs[k],
                                     neg_sems[k])

    for k in range(_NBUF):
        neg_copy(k, k).start()

    cq.wait()
    cr.wait()
    cp.wait()

    def add_rel(b, _):
        for s in range(_D // _L):
            sl = pl.ds(s * _L, _L)
            q_rows[b, sl] = q_rows[b, sl] + r_rows[b, sl]
        return 0
    lax.fori_loop(0, _BW, add_rel, 0)


    for g in range(_BW // _L):
        b_ids = g * _L + iota

        def pbody(d, acc):
            col = jnp.bitwise_and(d + iota, _D - 1)
            vals = plsc.load_gather(pos_rows, [b_ids, col])
            qv = plsc.load_gather(q_rows, [b_ids, col])
            return acc + jnp.abs(vals - qv)

        acc = lax.fori_loop(0, _D, pbody, jnp.zeros((_L,), jnp.float32),
                            unroll=4)
        out_pos_v[pl.ds(g * _L, _L)] = _GAMMA - acc

    n_grp = _NNEG // _L
    grp_rows = [g * _L + iota for g in range(n_grp)]

    def compute_batch(b, buf):
        bb = jnp.full((_L,), b, jnp.int32)

        def nbody(d, accs):
            col = jnp.bitwise_and(d + iota, _D - 1)
            qb = plsc.load_gather(q_rows, [bb, col])
            out = []
            for g in range(n_grp):
                vals = plsc.load_gather(buf, [grp_rows[g], col])
                out.append(accs[g] + jnp.abs(vals - qb))
            return tuple(out)

        accs = lax.fori_loop(
            0, _D, nbody,
            tuple(jnp.zeros((_L,), jnp.float32) for _ in range(n_grp)),
            unroll=2)
        for g in range(n_grp):
            out_neg_v[b, pl.ds(g * _L, _L)] = _GAMMA - accs[g]

    def outer(it, _):
        for k in range(_NBUF):
            b = it * _NBUF + k
            neg_copy(b, k).wait()
            compute_batch(b, bufs[k])

            @pl.when(b + _NBUF < _BW)
            def _():
                neg_copy(b + _NBUF, k).start()
        return 0

    lax.fori_loop(0, _BW // _NBUF, outer, 0)

    pltpu.sync_copy(out_pos_v, pos_out.at[pl.ds(base, _BW)])
    pltpu.sync_copy(out_neg_v, neg_out.at[pl.ds(base, _BW)])


@functools.cache
def _build():
    mesh = plsc.VectorSubcoreMesh(core_axis_name="c", subcore_axis_name="s")
    scratch = [
        pltpu.VMEM((_BW,), jnp.int32),
        pltpu.VMEM((_BW,), jnp.int32),
        pltpu.VMEM((_BW,), jnp.int32),
        pltpu.VMEM((_BW, _NNEG), jnp.int32),
        pltpu.VMEM((_BW, _D), jnp.float32),
        pltpu.VMEM((_BW, _D), jnp.float32),
        pltpu.VMEM((_BW, _D), jnp.float32),
        [pltpu.VMEM((_NNEG, _D), jnp.float32) for _ in range(_NBUF)],
        pltpu.VMEM((_BW,), jnp.float32),
        pltpu.VMEM((_BW, _NNEG), jnp.float32),
        pltpu.SemaphoreType.DMA,
        pltpu.SemaphoreType.DMA,
        pltpu.SemaphoreType.DMA,
        [pltpu.SemaphoreType.DMA for _ in range(_NBUF)],
    ]
    return pl.kernel(
        _gqe_body,
        out_type=(
            jax.ShapeDtypeStruct((_B,), jnp.float32),
            jax.ShapeDtypeStruct((_B, _NNEG), jnp.float32),
        ),
        mesh=mesh,
        scratch_types=scratch,
        compiler_params=pltpu.CompilerParams(needs_layout_passes=False,
                                             use_tc_tiling_on_sc=False),
    )


def kernel(entity_table, relation_table, positive_sample, negative_sample,
           q_entity, q_relation):
    tail = jnp.reshape(entity_table[_NCHUNK * _CW:], (_NTAIL * _D,))
    ent_flat = _build_tr()(entity_table.T, tail)
    ent_lin = jnp.reshape(ent_flat, (_NE, _D))
    return _build()(ent_lin, relation_table, positive_sample,
                    negative_sample, q_entity, q_relation)
